# baseline (speedup 1.0000x reference)
import functools

import numpy as np
import jax
import jax.numpy as jnp
from jax.experimental import pallas as pl
from jax.experimental.pallas import tpu as pltpu

B, S, D, H, K = 1, 2048, 1024, 16, 32
DH = D // H
LEVELS = 12
T = 256
NT = S // T
W = 34
NO = 2 * W + 1
S_PAD = 2304


def _routes_np():
    n, k = S, K
    t = ((np.arange(n, dtype=np.float32) + np.float32(0.5)) / np.float32(n)).astype(np.float32)
    c = np.zeros(n, dtype=np.float32)
    frac = t
    stopped = np.zeros(n, dtype=bool)
    for l in range(LEVELS):
        d = np.clip(np.floor(frac * np.float32(3.0)).astype(np.int32), 0, 2)
        frac = (frac * np.float32(3.0) - d.astype(np.float32)).astype(np.float32)
        scale = np.float32(0.5 ** (l + 1))
        add = np.where(d == 1, np.float32(1.0), d.astype(np.float32) * np.float32(0.5)) * scale
        c = (c + np.where(stopped, np.float32(0.0), add).astype(np.float32)).astype(np.float32)
        stopped = stopped | (d == 1)
    pos = np.arange(n, dtype=np.float32)
    dist = (np.abs(c[:, None] - c[None, :]).astype(np.float32)
            + (np.abs(pos[:, None] - pos[None, :]) / np.float32(n * 1e6)).astype(np.float32))
    return np.argsort(dist.astype(np.float32), axis=-1, kind="stable")[:, :k].astype(np.int32)


NG = 9


@functools.lru_cache(maxsize=1)
def _static_tables():
    routes = _routes_np()
    off = routes - np.arange(S, dtype=np.int32)[:, None]
    assert np.abs(off).max() <= W
    val = np.zeros((S, NO), dtype=np.float32)
    np.put_along_axis(val, off + W, 1.0, axis=1)
    assert (val[:, W] == 1.0).all()
    v2p = np.zeros((NT, NG * T, 8 * H), dtype=np.float32)
    for g in range(NG):
        for a in range(8):
            o = 8 * a + g if g < 8 else 64 + a
            if o >= NO:
                continue
            v2p[:, g * T:(g + 1) * T, 16 * a:16 * (a + 1)] = (
                val[:, o].reshape(NT, T, 1))
    rep = np.kron(np.eye(H, dtype=np.float32), np.ones((1, DH), np.float32))
    return v2p, rep


def _proj_kernel(x_ref, win_ref, bdqn_ref, b1_ref,
                 h_ref, q_ref, n_ref):
    xv = x_ref[...]
    h = jnp.dot(xv, win_ref[...], preferred_element_type=jnp.float32)
    h_ref[...] = h.astype(jnp.bfloat16)
    qn = jnp.dot(h.astype(jnp.bfloat16), bdqn_ref[...],
                 preferred_element_type=jnp.float32)
    q_ref[...] = (qn[:, :D] + b1_ref[...]).astype(jnp.bfloat16)
    n_ref[...] = qn[:, D:].astype(jnp.bfloat16)


LW = T + 64


def _fuse_kernel(q_ref, x_ref, nb0_ref, nb1_ref, hb0_ref, hb1_ref, m2_ref,
                 wo_ref, w2f_ref, w2l_ref, rbd_ref, r_ref, bo_ref,
                 out_ref, z_scr):
    qv = q_ref[...]
    nwin = jnp.concatenate([nb0_ref[...], nb1_ref[...]], axis=0)

    for g in range(NG):
        if g < 8:
            nr = nwin[g:g + LW]
            pieces = [jnp.maximum(qv + nr[8 * a:8 * a + T], jnp.bfloat16(0.0))
                      for a in range(8)]
        else:
            p5 = [jnp.maximum(qv + nwin[64 + r:64 + r + T], jnp.bfloat16(0.0))
                  for r in range(5)]
            pieces = p5 + [p5[0]] * 3
        tcat = jnp.concatenate(pieces, axis=1)
        wg = w2f_ref[...] if g < 8 else w2l_ref[...]
        z_scr[g] = jnp.dot(tcat, wg, preferred_element_type=jnp.float32)

    zs = z_scr[2][:, 64:80]
    zs_tile = jnp.concatenate([zs] * 8, axis=1)

    hwin = jnp.concatenate([hb0_ref[...], hb1_ref[...]], axis=0)
    acc = jnp.zeros((T, D), jnp.float32)
    l8 = jnp.zeros((T, 8 * H), jnp.float32)
    for g in range(NG):
        v2g = m2_ref[0, g * T:(g + 1) * T, :]
        E = jnp.exp(z_scr[g] - zs_tile) * v2g
        l8 = l8 + E
        PR = jnp.dot(E.astype(jnp.bfloat16), rbd_ref[...],
                     preferred_element_type=jnp.float32)
        if g < 8:
            hr = hwin[g:g + LW]
            for a in range(8):
                acc = acc + PR[:, D * a:D * (a + 1)] * hr[8 * a:8 * a + T]
        else:
            for a in range(5):
                acc = acc + (PR[:, D * a:D * (a + 1)]
                             * hwin[64 + a:64 + a + T])

    l = l8[:, :H]
    for a in range(1, 8):
        l = l + l8[:, 16 * a:16 * (a + 1)]
    lrep = jnp.dot(l, r_ref[...], preferred_element_type=jnp.float32)
    fused = acc / lrep
    out = jnp.dot(fused.astype(jnp.bfloat16), wo_ref[...],
                  preferred_element_type=jnp.float32)
    out_ref[...] = out + bo_ref[...] + x_ref[...]


def kernel(x, W_in, W1q, W1n, b1, W2, b2, W_out, b_out):
    inv_np, rep_np = _static_tables()
    x2 = x.reshape(S, D)
    eye = jnp.eye(H, dtype=jnp.float32)
    bdqn = jnp.concatenate(
        [jnp.kron(eye, W1q), jnp.kron(eye, W1n)], axis=1
    ).astype(jnp.bfloat16)
    w2bd = jnp.kron(eye, W2)
    eye8 = jnp.eye(8, dtype=jnp.float32)
    w2f = jnp.kron(eye8, w2bd).astype(jnp.bfloat16)
    left = jnp.diag(jnp.asarray([1.0] * 5 + [0.0] * 3, jnp.float32))
    w2l = jnp.kron(left, w2bd).astype(jnp.bfloat16)
    b1t = jnp.tile(b1, H).reshape(1, D)
    bo = (b_out + b2[0] * 0.0).reshape(1, D)
    m2 = jnp.asarray(inv_np)
    rep = jnp.asarray(rep_np).astype(jnp.bfloat16)
    rbd = jnp.kron(eye8, jnp.asarray(rep_np)).astype(jnp.bfloat16)

    h, q, n = pl.pallas_call(
        _proj_kernel,
        grid=(NT,),
        in_specs=[
            pl.BlockSpec((T, D), lambda i: (i, 0)),
            pl.BlockSpec((D, D), lambda i: (0, 0)),
            pl.BlockSpec((D, 2 * D), lambda i: (0, 0)),
            pl.BlockSpec((1, D), lambda i: (0, 0)),
        ],
        out_specs=[
            pl.BlockSpec((T, D), lambda i: (i, 0)),
            pl.BlockSpec((T, D), lambda i: (i, 0)),
            pl.BlockSpec((T, D), lambda i: (i, 0)),
        ],
        out_shape=[jax.ShapeDtypeStruct((S, D), jnp.bfloat16)] * 3,
    )(x2, W_in, bdqn, b1t)

    h_pad = jnp.pad(h, ((W, S_PAD - S - W), (0, 0)))
    n_pad = jnp.pad(n, ((W, S_PAD - S - W), (0, 0)))

    out2 = pl.pallas_call(
        _fuse_kernel,
        grid=(NT,),
        in_specs=[
            pl.BlockSpec((T, D), lambda i: (i, 0)),
            pl.BlockSpec((T, D), lambda i: (i, 0)),
            pl.BlockSpec((T, D), lambda i: (i, 0)),
            pl.BlockSpec((T, D), lambda i: (i + 1, 0)),
            pl.BlockSpec((T, D), lambda i: (i, 0)),
            pl.BlockSpec((T, D), lambda i: (i + 1, 0)),
            pl.BlockSpec((1, NG * T, 8 * H), lambda i: (i, 0, 0)),
            pl.BlockSpec((D, D), lambda i: (0, 0)),
            pl.BlockSpec((8 * D, 8 * H), lambda i: (0, 0)),
            pl.BlockSpec((8 * D, 8 * H), lambda i: (0, 0)),
            pl.BlockSpec((8 * H, 8 * D), lambda i: (0, 0)),
            pl.BlockSpec((H, D), lambda i: (0, 0)),
            pl.BlockSpec((1, D), lambda i: (0, 0)),
        ],
        out_specs=pl.BlockSpec((T, D), lambda i: (i, 0)),
        out_shape=jax.ShapeDtypeStruct((S, D), jnp.float32),
        scratch_shapes=[
            pltpu.VMEM((NG, T, 8 * H), jnp.float32),
        ],
    )(q, x2, n_pad, n_pad, h_pad, h_pad, m2, W_out.astype(jnp.bfloat16),
      w2f, w2l, rbd, rep, bo)

    return out2.reshape(B, S, D)


# speedup vs baseline: 1.0379x; 1.0379x over previous
import functools

import numpy as np
import jax
import jax.numpy as jnp
from jax.experimental import pallas as pl
from jax.experimental.pallas import tpu as pltpu

B, S, D, H, K = 1, 2048, 1024, 16, 32
DH = D // H
LEVELS = 12
T = 256
NT = S // T
W = 34
NO = 2 * W + 1
S_PAD = 2304


def _routes_np():
    n, k = S, K
    t = ((np.arange(n, dtype=np.float32) + np.float32(0.5)) / np.float32(n)).astype(np.float32)
    c = np.zeros(n, dtype=np.float32)
    frac = t
    stopped = np.zeros(n, dtype=bool)
    for l in range(LEVELS):
        d = np.clip(np.floor(frac * np.float32(3.0)).astype(np.int32), 0, 2)
        frac = (frac * np.float32(3.0) - d.astype(np.float32)).astype(np.float32)
        scale = np.float32(0.5 ** (l + 1))
        add = np.where(d == 1, np.float32(1.0), d.astype(np.float32) * np.float32(0.5)) * scale
        c = (c + np.where(stopped, np.float32(0.0), add).astype(np.float32)).astype(np.float32)
        stopped = stopped | (d == 1)
    pos = np.arange(n, dtype=np.float32)
    dist = (np.abs(c[:, None] - c[None, :]).astype(np.float32)
            + (np.abs(pos[:, None] - pos[None, :]) / np.float32(n * 1e6)).astype(np.float32))
    return np.argsort(dist.astype(np.float32), axis=-1, kind="stable")[:, :k].astype(np.int32)


@functools.lru_cache(maxsize=1)
def _static_tables():
    routes = _routes_np()
    off = routes - np.arange(S, dtype=np.int32)[:, None]
    assert np.abs(off).max() <= W
    val = np.zeros((S, NO), dtype=np.float32)
    np.put_along_axis(val, off + W, 1.0, axis=1)
    assert (val[:, W] == 1.0).all()
    inv = val.reshape(NT, T, NO)
    rep = np.kron(np.eye(H, dtype=np.float32), np.ones((1, DH), np.float32))
    return inv, rep


def _proj_kernel(x_ref, win_ref, bdqn_ref, b1_ref,
                 h_ref, q_ref, n_ref):
    xv = x_ref[...]
    h = jnp.dot(xv, win_ref[...], preferred_element_type=jnp.float32)
    h_ref[...] = h.astype(jnp.bfloat16)
    qn = jnp.dot(h.astype(jnp.bfloat16), bdqn_ref[...],
                 preferred_element_type=jnp.float32)
    q_ref[...] = (qn[:, :D] + b1_ref[...]).astype(jnp.bfloat16)
    n_ref[...] = qn[:, D:].astype(jnp.bfloat16)


LW = T + 64


def _fuse_kernel(q_ref, x_ref, nb0_ref, nb1_ref, hb0_ref, hb1_ref, m2_ref,
                 wo_ref, w2_ref, r_ref, bo_ref, out_ref, z_scr):
    qv = q_ref[...]
    v2 = m2_ref[0]
    nwin = jnp.concatenate([nb0_ref[...], nb1_ref[...]], axis=0)

    for r in range(8):
        nr = nwin[r:r + LW]
        for a in range((NO - r + 7) // 8):
            o = 8 * a + r
            t = jnp.maximum(qv + nr[8 * a:8 * a + T], jnp.bfloat16(0.0))
            z_scr[o] = jnp.dot(t, w2_ref[...],
                               preferred_element_type=jnp.float32).astype(jnp.bfloat16)

    z_self = z_scr[W].astype(jnp.float32)
    hwin = jnp.concatenate([hb0_ref[...], hb1_ref[...]], axis=0)
    acc = jnp.zeros((T, D), jnp.float32)
    l = jnp.zeros((T, H), jnp.float32)
    for r in range(8):
        hr = hwin[r:r + LW]
        for a in range((NO - r + 7) // 8):
            o = 8 * a + r
            e = jnp.exp(z_scr[o].astype(jnp.float32) - z_self) * v2[:, o:o + 1]
            pr = jnp.dot(e.astype(jnp.bfloat16), r_ref[...],
                         preferred_element_type=jnp.float32)
            acc = acc + pr * hr[8 * a:8 * a + T]
            l = l + e

    lrep = jnp.dot(l, r_ref[...], preferred_element_type=jnp.float32)
    fused = acc / lrep
    out = jnp.dot(fused.astype(jnp.bfloat16), wo_ref[...],
                  preferred_element_type=jnp.float32)
    out_ref[...] = out + bo_ref[...] + x_ref[...]


def kernel(x, W_in, W1q, W1n, b1, W2, b2, W_out, b_out):
    inv_np, rep_np = _static_tables()
    x2 = x.reshape(S, D)
    eye = jnp.eye(H, dtype=jnp.float32)
    bdqn = jnp.concatenate(
        [jnp.kron(eye, W1q), jnp.kron(eye, W1n)], axis=1
    ).astype(jnp.bfloat16)
    w2bd = jnp.kron(eye, W2).astype(jnp.bfloat16)
    b1t = jnp.tile(b1, H).reshape(1, D)
    bo = (b_out + b2[0] * 0.0).reshape(1, D)
    m2 = jnp.asarray(inv_np)
    rep = jnp.asarray(rep_np).astype(jnp.bfloat16)

    h, q, n = pl.pallas_call(
        _proj_kernel,
        grid=(NT,),
        in_specs=[
            pl.BlockSpec((T, D), lambda i: (i, 0)),
            pl.BlockSpec((D, D), lambda i: (0, 0)),
            pl.BlockSpec((D, 2 * D), lambda i: (0, 0)),
            pl.BlockSpec((1, D), lambda i: (0, 0)),
        ],
        out_specs=[
            pl.BlockSpec((T, D), lambda i: (i, 0)),
            pl.BlockSpec((T, D), lambda i: (i, 0)),
            pl.BlockSpec((T, D), lambda i: (i, 0)),
        ],
        out_shape=[jax.ShapeDtypeStruct((S, D), jnp.bfloat16)] * 3,
    )(x2, W_in, bdqn, b1t)

    h_pad = jnp.pad(h, ((W, S_PAD - S - W), (0, 0)))
    n_pad = jnp.pad(n, ((W, S_PAD - S - W), (0, 0)))

    out2 = pl.pallas_call(
        _fuse_kernel,
        grid=(NT,),
        in_specs=[
            pl.BlockSpec((T, D), lambda i: (i, 0)),
            pl.BlockSpec((T, D), lambda i: (i, 0)),
            pl.BlockSpec((T, D), lambda i: (i, 0)),
            pl.BlockSpec((T, D), lambda i: (i + 1, 0)),
            pl.BlockSpec((T, D), lambda i: (i, 0)),
            pl.BlockSpec((T, D), lambda i: (i + 1, 0)),
            pl.BlockSpec((1, T, NO), lambda i: (i, 0, 0)),
            pl.BlockSpec((D, D), lambda i: (0, 0)),
            pl.BlockSpec((D, H), lambda i: (0, 0)),
            pl.BlockSpec((H, D), lambda i: (0, 0)),
            pl.BlockSpec((1, D), lambda i: (0, 0)),
        ],
        out_specs=pl.BlockSpec((T, D), lambda i: (i, 0)),
        out_shape=jax.ShapeDtypeStruct((S, D), jnp.float32),
        scratch_shapes=[
            pltpu.VMEM((NO, T, H), jnp.bfloat16),
        ],
    )(q, x2, n_pad, n_pad, h_pad, h_pad, m2, W_out.astype(jnp.bfloat16),
      w2bd, rep, bo)

    return out2.reshape(B, S, D)
